# baseline (device time: 50587 ns/iter reference)
import numpy as np
import jax
import jax.numpy as jnp
from jax import lax
from jax.experimental import pallas as pl
from jax.experimental.pallas import tpu as pltpu

N_DEV = 4
B, SQ, D = 2, 256, 768
H_LOC, DH = 4, 64


def _rope_consts():
    inv = 1.0 / (10000.0 ** (np.arange(0, DH, 2) / DH))
    pos = np.arange(SQ)[:, None] * inv[None, :]
    cos = np.repeat(np.cos(pos), 2, axis=-1).astype(np.float32)
    sin = np.repeat(np.sin(pos), 2, axis=-1).astype(np.float32)
    R = np.zeros((DH, DH), dtype=np.float32)
    for k in range(DH // 2):
        R[2 * k + 1, 2 * k] = -1.0
        R[2 * k, 2 * k + 1] = 1.0
    return cos, sin, R


def kernel(x, Wq, Wk, Wv, Wo):
    cos_np, sin_np, R_np = _rope_consts()
    cos_c = jnp.asarray(cos_np)
    sin_c = jnp.asarray(sin_np)
    R_c = jnp.asarray(R_np)

    def body(x_ref, wq_ref, wk_ref, wv_ref, wo_ref, cos_ref, sin_ref, r_ref,
             out_ref, comm_ref, ctx_ref, send_sems, recv_sems):
        my = lax.axis_index("i")
        left = lax.rem(my + N_DEV - 1, N_DEV)
        right = lax.rem(my + 1, N_DEV)

        barrier_sem = pltpu.get_barrier_semaphore()
        for nbr in (left, right):
            pl.semaphore_signal(
                barrier_sem, inc=1,
                device_id=(nbr,), device_id_type=pl.DeviceIdType.MESH,
            )
        pl.semaphore_wait(barrier_sem, 2)

        f32 = jnp.float32
        cos = cos_ref[...]
        sin = sin_ref[...]
        R = r_ref[...]
        wq = wq_ref[...].astype(jnp.bfloat16)
        wk = wk_ref[...].astype(jnp.bfloat16)
        wv = wv_ref[...].astype(jnp.bfloat16)

        for b in range(B):
            xb = x_ref[b].astype(jnp.bfloat16)
            q = jnp.dot(xb, wq, preferred_element_type=f32)
            k = jnp.dot(xb, wk, preferred_element_type=f32)
            v = jnp.dot(xb, wv, preferred_element_type=f32)
            for h in range(H_LOC):
                sl = slice(h * DH, (h + 1) * DH)
                qh = q[:, sl]
                kh = k[:, sl]
                qh = qh * cos + jnp.dot(qh, R, preferred_element_type=f32) * sin
                kh = kh * cos + jnp.dot(kh, R, preferred_element_type=f32) * sin
                s = lax.dot_general(
                    qh.astype(jnp.bfloat16), kh.astype(jnp.bfloat16),
                    (((1,), (1,)), ((), ())),
                    preferred_element_type=f32,
                ) * 0.125
                s = s - jnp.max(s, axis=-1, keepdims=True)
                w = jnp.exp(s)
                w = w / jnp.sum(w, axis=-1, keepdims=True)
                ctx = jnp.dot(
                    w.astype(jnp.bfloat16), v[:, sl].astype(jnp.bfloat16),
                    preferred_element_type=f32,
                )
                ctx_ref[b, :, sl] = ctx.astype(jnp.bfloat16)

        wo = wo_ref[...].astype(jnp.bfloat16)
        for b in range(B):
            pb = jnp.dot(ctx_ref[b], wo, preferred_element_type=f32)
            out_ref[b] = pb
            comm_ref[0, b] = pb.astype(jnp.bfloat16)

        for hop in range(N_DEV - 1):
            send_slot = hop % 2
            recv_slot = (hop + 1) % 2
            rdma = pltpu.make_async_remote_copy(
                src_ref=comm_ref.at[send_slot],
                dst_ref=comm_ref.at[recv_slot],
                send_sem=send_sems.at[send_slot],
                recv_sem=recv_sems.at[recv_slot],
                device_id=(right,),
                device_id_type=pl.DeviceIdType.MESH,
            )
            rdma.start()
            rdma.wait()
            out_ref[...] = out_ref[...] + comm_ref[recv_slot].astype(f32)

    return pl.pallas_call(
        body,
        out_shape=jax.ShapeDtypeStruct((B, SQ, D), jnp.float32),
        in_specs=[pl.BlockSpec(memory_space=pltpu.VMEM)] * 8,
        out_specs=pl.BlockSpec(memory_space=pltpu.VMEM),
        scratch_shapes=[
            pltpu.VMEM((2, B, SQ, D), jnp.bfloat16),
            pltpu.VMEM((B, SQ, H_LOC * DH), jnp.bfloat16),
            pltpu.SemaphoreType.DMA((2,)),
            pltpu.SemaphoreType.DMA((2,)),
        ],
        compiler_params=pltpu.CompilerParams(collective_id=0),
    )(x, Wq, Wk, Wv, Wo, cos_c, sin_c, R_c)


# device time: 29948 ns/iter; 1.6892x vs baseline; 1.6892x over previous
import numpy as np
import jax
import jax.numpy as jnp
from jax import lax
from jax.experimental import pallas as pl
from jax.experimental.pallas import tpu as pltpu

N_DEV = 4
B, SQ, D = 2, 256, 768
H_LOC, DH = 4, 64
ROWS = B * SQ
CH = ROWS // N_DEV


def _rope_consts():
    inv = 1.0 / (10000.0 ** (np.arange(0, DH, 2) / DH))
    pos = np.arange(SQ)[:, None] * inv[None, :]
    cos = np.repeat(np.cos(pos), 2, axis=-1).astype(np.float32)
    sin = np.repeat(np.sin(pos), 2, axis=-1).astype(np.float32)
    R = np.zeros((DH, DH), dtype=np.float32)
    for k in range(DH // 2):
        R[2 * k + 1, 2 * k] = -1.0
        R[2 * k, 2 * k + 1] = 1.0
    return cos, sin, R


def kernel(x, Wq, Wk, Wv, Wo):
    cos_np, sin_np, R_np = _rope_consts()
    cos_c = jnp.asarray(cos_np)
    sin_c = jnp.asarray(sin_np)
    R_c = jnp.asarray(R_np)

    def body(x_ref, wq_ref, wk_ref, wv_ref, wo_ref, cos_ref, sin_ref, r_ref,
             out_ref, sbuf, ctx_ref, agsrc, rs_buf, ag_buf,
             rs_ssems, rs_rsems, ag_ssems, ag_rsems):
        my = lax.axis_index("i")

        barrier_sem = pltpu.get_barrier_semaphore()
        for d in range(1, N_DEV):
            pl.semaphore_signal(
                barrier_sem, inc=1,
                device_id=(lax.rem(my + d, N_DEV),),
                device_id_type=pl.DeviceIdType.MESH,
            )
        pl.semaphore_wait(barrier_sem, N_DEV - 1)

        f32 = jnp.float32
        bf16 = jnp.bfloat16
        cos = cos_ref[...]
        sin = sin_ref[...]
        R = r_ref[...]
        wq = wq_ref[...].astype(bf16)
        wk = wk_ref[...].astype(bf16)
        wv = wv_ref[...].astype(bf16)

        for b in range(B):
            xb = x_ref[b].astype(bf16)
            q = jnp.dot(xb, wq, preferred_element_type=f32)
            k = jnp.dot(xb, wk, preferred_element_type=f32)
            v = jnp.dot(xb, wv, preferred_element_type=f32)
            for h in range(H_LOC):
                sl = slice(h * DH, (h + 1) * DH)
                qh = q[:, sl]
                kh = k[:, sl]
                qh = qh * cos + jnp.dot(qh, R, preferred_element_type=f32) * sin
                kh = kh * cos + jnp.dot(kh, R, preferred_element_type=f32) * sin
                s = lax.dot_general(
                    qh.astype(bf16), kh.astype(bf16),
                    (((1,), (1,)), ((), ())),
                    preferred_element_type=f32,
                ) * 0.125
                s = s - jnp.max(s, axis=-1, keepdims=True)
                w = jnp.exp(s)
                w = w / jnp.sum(w, axis=-1, keepdims=True)
                ctx = jnp.dot(
                    w.astype(bf16), v[:, sl].astype(bf16),
                    preferred_element_type=f32,
                )
                ctx_ref[b, :, sl] = ctx.astype(bf16)

        wo = wo_ref[...].astype(bf16)
        for b in range(B):
            pb = jnp.dot(ctx_ref[b], wo, preferred_element_type=f32)
            out_ref[b] = pb
            sbuf[b * SQ:(b + 1) * SQ, :] = pb.astype(bf16)

        sends = []
        for d in range(1, N_DEV):
            t = lax.rem(my + d, N_DEV)
            rdma = pltpu.make_async_remote_copy(
                src_ref=sbuf.at[pl.ds(t * CH, CH), :],
                dst_ref=rs_buf.at[N_DEV - 1 - d],
                send_sem=rs_ssems.at[d - 1],
                recv_sem=rs_rsems.at[N_DEV - 1 - d],
                device_id=(t,),
                device_id_type=pl.DeviceIdType.MESH,
            )
            rdma.start()
            sends.append(rdma)

        for s in range(N_DEV - 1):
            recv = pltpu.make_async_remote_copy(
                src_ref=rs_buf.at[s],
                dst_ref=rs_buf.at[s],
                send_sem=rs_ssems.at[s],
                recv_sem=rs_rsems.at[s],
                device_id=(my,),
                device_id_type=pl.DeviceIdType.MESH,
            )
            recv.wait_recv()

        b_my = lax.div(my, 2)
        r_my = lax.rem(my, 2) * CH
        red = out_ref[b_my, pl.ds(r_my, CH), :]
        for s in range(N_DEV - 1):
            red = red + rs_buf[s].astype(f32)
        out_ref[b_my, pl.ds(r_my, CH), :] = red
        agsrc[...] = red.astype(bf16)

        for d in range(1, N_DEV):
            t = lax.rem(my + d, N_DEV)
            rdma = pltpu.make_async_remote_copy(
                src_ref=agsrc,
                dst_ref=ag_buf.at[N_DEV - 1 - d],
                send_sem=ag_ssems.at[d - 1],
                recv_sem=ag_rsems.at[N_DEV - 1 - d],
                device_id=(t,),
                device_id_type=pl.DeviceIdType.MESH,
            )
            rdma.start()
            sends.append(rdma)

        for d in range(1, N_DEV):
            s = N_DEV - 1 - d
            recv = pltpu.make_async_remote_copy(
                src_ref=ag_buf.at[s],
                dst_ref=ag_buf.at[s],
                send_sem=ag_ssems.at[s],
                recv_sem=ag_rsems.at[s],
                device_id=(my,),
                device_id_type=pl.DeviceIdType.MESH,
            )
            recv.wait_recv()
            c = lax.rem(my + N_DEV - d, N_DEV)
            out_ref[lax.div(c, 2), pl.ds(lax.rem(c, 2) * CH, CH), :] = (
                ag_buf[s].astype(f32)
            )

        for rdma in sends:
            rdma.wait_send()

    return pl.pallas_call(
        body,
        out_shape=jax.ShapeDtypeStruct((B, SQ, D), jnp.float32),
        in_specs=[pl.BlockSpec(memory_space=pltpu.VMEM)] * 8,
        out_specs=pl.BlockSpec(memory_space=pltpu.VMEM),
        scratch_shapes=[
            pltpu.VMEM((ROWS, D), jnp.bfloat16),
            pltpu.VMEM((B, SQ, H_LOC * DH), jnp.bfloat16),
            pltpu.VMEM((CH, D), jnp.bfloat16),
            pltpu.VMEM((N_DEV - 1, CH, D), jnp.bfloat16),
            pltpu.VMEM((N_DEV - 1, CH, D), jnp.bfloat16),
            pltpu.SemaphoreType.DMA((N_DEV - 1,)),
            pltpu.SemaphoreType.DMA((N_DEV - 1,)),
            pltpu.SemaphoreType.DMA((N_DEV - 1,)),
            pltpu.SemaphoreType.DMA((N_DEV - 1,)),
        ],
        compiler_params=pltpu.CompilerParams(collective_id=0),
    )(x, Wq, Wk, Wv, Wo, cos_c, sin_c, R_c)
